# baseline (speedup 1.0000x reference)
import functools

import jax
import jax.numpy as jnp
from jax import lax
from jax.experimental import pallas as pl
from jax.experimental.pallas import tpu as pltpu
from jax.experimental.pallas import tpu_sc as plsc

N_FIELDS = 26
VOCAB = 100000
EMB_DIM = 16
NUM_DIM = 13
BATCH = 16384

VPAD = 100096
TAIL0 = 99968

NUM_WORKERS = 32
N_CHUNKS = 16
CHUNK = BATCH // N_CHUNKS
TASKS_PER_TILE = (N_FIELDS * N_CHUNKS) // NUM_WORKERS

BULK_CHUNKS = 97
TOT_BULK = N_FIELDS * BULK_CHUNKS
BULK_STEPS = (TOT_BULK + NUM_WORKERS - 1) // NUM_WORKERS


def _tail_body(tab_ref, out_ref):
    x = tab_ref[0]
    eye = jnp.eye(EMB_DIM, dtype=jnp.float32)
    y = jax.lax.dot_general(x, eye, (((0,), (0,)), ((), ())),
                            preferred_element_type=jnp.float32)
    y8 = y.reshape(16, 8, EMB_DIM)
    out_ref[...] = jnp.concatenate([y8[:, h, :] for h in range(8)], axis=1)


def _tail(tab_t):
    return pl.pallas_call(
        _tail_body,
        grid=(N_FIELDS,),
        in_specs=[pl.BlockSpec((1, EMB_DIM, 128),
                               lambda f: (f, 0, TAIL0 // 128))],
        out_specs=pl.BlockSpec((16, 128), lambda f: (f, 0)),
        out_shape=jax.ShapeDtypeStruct((N_FIELDS * 16, 128), jnp.float32),
    )(tab_t)


def _reformat_body(tab_hbm, tail_hbm, cats_hbm, ltab_hbm, lcats_hbm,
                   inb0, inb1, outb0, outb1, rowb, tailb,
                   sin0, sin1, sout0, sout1):
    wid = lax.axis_index("s") * 2 + lax.axis_index("c")
    lanes = lax.iota(jnp.int32, 16)

    def chunk_q(t):
        return jnp.minimum(t * NUM_WORKERS + wid, TOT_BULK - 1)

    def start_in(t, ib, sem):
        q = chunk_q(t)
        f = q // BULK_CHUNKS
        col0 = (q % BULK_CHUNKS) * 1024
        pltpu.async_copy(tab_hbm.at[f, :, pl.ds(col0, 1024)], ib, sem)

    def wait_in(sem):
        pltpu.make_async_copy(tab_hbm.at[0, :, pl.ds(0, 1024)], inb0,
                              sem).wait()

    def extract(ib, ob, width):
        @plsc.parallel_loop(0, width, unroll=32)
        def _(v):
            x = plsc.load_gather(ib, [lanes, jnp.full((16,), v, jnp.int32)])
            ob[pl.ds(v * EMB_DIM, EMB_DIM)] = x

    def start_out(t, ob, sem):
        q = chunk_q(t)
        f = q // BULK_CHUNKS
        col0 = (q % BULK_CHUNKS) * 1024
        dst = (f * VPAD + col0) * EMB_DIM
        pltpu.async_copy(ob, ltab_hbm.at[pl.ds(dst, 1024 * EMB_DIM)], sem)

    def wait_out(sem):
        pltpu.make_async_copy(outb0, ltab_hbm.at[pl.ds(0, 1024 * EMB_DIM)],
                              sem).wait()

    start_in(0, inb0, sin0)

    def phase(t, ib, ob, sin, sout, ib_next, sin_next):
        @pl.when(t + 1 < BULK_STEPS)
        def _():
            start_in(t + 1, ib_next, sin_next)

        wait_in(sin)

        @pl.when(t >= 2)
        def _():
            wait_out(sout)

        extract(ib, ob, 1024)
        start_out(t, ob, sout)

    def body(t, _):
        @pl.when(t % 2 == 0)
        def _():
            phase(t, inb0, outb0, sin0, sout0, inb1, sin1)

        @pl.when(t % 2 == 1)
        def _():
            phase(t, inb1, outb1, sin1, sout1, inb0, sin0)

        return 0

    lax.fori_loop(0, BULK_STEPS, body, 0)
    wait_out(sout0)
    wait_out(sout1)

    @pl.when(wid < N_FIELDS)
    def _():
        f = wid
        w = TAIL0 - BULK_CHUNKS * 1024
        pltpu.async_copy(tab_hbm.at[f, :, pl.ds(BULK_CHUNKS * 1024, w)],
                         inb0.at[:, pl.ds(0, w)], sin0)
        pltpu.make_async_copy(tab_hbm.at[0, :, pl.ds(0, w)],
                              inb0.at[:, pl.ds(0, w)], sin0).wait()
        extract(inb0, outb0, w)
        pltpu.async_copy(outb0.at[pl.ds(0, w * EMB_DIM)],
                         ltab_hbm.at[pl.ds((f * VPAD + BULK_CHUNKS * 1024)
                                           * EMB_DIM, w * EMB_DIM)], sout0)
        pltpu.sync_copy(tail_hbm.at[pl.ds(f * 16, 16)], tailb)

        def trow(v, _):
            x = plsc.load_gather(
                tailb, [jnp.full((16,), v // 8, jnp.int32),
                        (v % 8) * EMB_DIM + lanes])
            outb1[pl.ds(v * EMB_DIM, EMB_DIM)] = x
            return 0

        lax.fori_loop(0, 128, trow, 0, unroll=8)
        pltpu.async_copy(outb1.at[pl.ds(0, 128 * EMB_DIM)],
                         ltab_hbm.at[pl.ds((f * VPAD + TAIL0) * EMB_DIM,
                                           128 * EMB_DIM)], sout1)
        pltpu.sync_copy(cats_hbm.at[f], rowb)

        def add_off(k, _):
            rowb[pl.ds(k * 16, 16)] = rowb[pl.ds(k * 16, 16)] + f * VPAD
            return 0

        lax.fori_loop(0, BATCH // 16, add_off, 0, unroll=8)
        pltpu.sync_copy(rowb, lcats_hbm.at[pl.ds(f * BATCH, BATCH)])
        pltpu.make_async_copy(outb0.at[pl.ds(0, w * EMB_DIM)],
                              ltab_hbm.at[pl.ds(0, w * EMB_DIM)],
                              sout0).wait()
        pltpu.make_async_copy(outb1.at[pl.ds(0, 128 * EMB_DIM)],
                              ltab_hbm.at[pl.ds(0, 128 * EMB_DIM)],
                              sout1).wait()


_reformat = functools.partial(
    pl.kernel,
    out_type=(
        jax.ShapeDtypeStruct((N_FIELDS * VPAD * EMB_DIM,), jnp.float32),
        jax.ShapeDtypeStruct((N_FIELDS * BATCH,), jnp.int32),
    ),
    mesh=plsc.VectorSubcoreMesh(core_axis_name="c", subcore_axis_name="s"),
    scratch_types=[
        pltpu.VMEM((EMB_DIM, 1024), jnp.float32),
        pltpu.VMEM((EMB_DIM, 1024), jnp.float32),
        pltpu.VMEM((1024 * EMB_DIM,), jnp.float32),
        pltpu.VMEM((1024 * EMB_DIM,), jnp.float32),
        pltpu.VMEM((BATCH,), jnp.int32),
        pltpu.VMEM((16, 128), jnp.float32),
        pltpu.SemaphoreType.DMA,
        pltpu.SemaphoreType.DMA,
        pltpu.SemaphoreType.DMA,
        pltpu.SemaphoreType.DMA,
    ],
    compiler_params=pltpu.CompilerParams(needs_layout_passes=False),
)(_reformat_body)


def _gather_body(lcats_hbm, ltab_hbm, out_hbm, idx_v, rows_v, sem):
    wid = lax.axis_index("s") * 2 + lax.axis_index("c")

    def body(t, carry):
        task = wid * TASKS_PER_TILE + t
        f = task // N_CHUNKS
        c = task % N_CHUNKS
        base = c * CHUNK
        pltpu.sync_copy(lcats_hbm.at[pl.ds(f * BATCH + base, CHUNK)], idx_v)
        pltpu.async_copy(ltab_hbm.at[idx_v], rows_v, sem).wait()
        pltpu.sync_copy(rows_v,
                        out_hbm.at[f // 8, pl.ds(base, CHUNK),
                                   pl.ds((f % 8) * EMB_DIM, EMB_DIM)])
        return carry

    lax.fori_loop(0, TASKS_PER_TILE, body, 0)


_gather = functools.partial(
    pl.kernel,
    out_type=jax.ShapeDtypeStruct((4, BATCH, 128), jnp.float32),
    mesh=plsc.VectorSubcoreMesh(core_axis_name="c", subcore_axis_name="s"),
    scratch_types=[
        pltpu.VMEM((CHUNK,), jnp.int32),
        pltpu.VMEM((CHUNK, EMB_DIM), jnp.float32),
        pltpu.SemaphoreType.DMA,
    ],
    compiler_params=pltpu.CompilerParams(use_tc_tiling_on_sc=False),
)(_gather_body)


def _mlp_body(cat_ref, nums_ref, w1e_ref, w1n_ref, b1_ref, w2_ref, b2_ref,
              out_ref):
    x4 = cat_ref[...]
    w1e = w1e_ref[...]
    acc = nums_ref[...] @ w1n_ref[...] + b1_ref[...]
    for g in range(3):
        acc = acc + x4[g] @ w1e[g * 128:(g + 1) * 128]
    acc = acc + x4[3][:, :32] @ w1e[384:416]
    h = jnp.maximum(acc, 0.0)
    out_ref[...] = h @ w2_ref[...] + b2_ref[...]


def _mlp(cats4, nums, w1e, w1n, b1, w2, b2):
    bm = 2048
    grid = BATCH // bm
    return pl.pallas_call(
        _mlp_body,
        grid=(grid,),
        in_specs=[
            pl.BlockSpec((4, bm, 128), lambda i: (0, i, 0)),
            pl.BlockSpec((bm, NUM_DIM), lambda i: (i, 0)),
            pl.BlockSpec((N_FIELDS * EMB_DIM, 64), lambda i: (0, 0)),
            pl.BlockSpec((NUM_DIM, 64), lambda i: (0, 0)),
            pl.BlockSpec((1, 64), lambda i: (0, 0)),
            pl.BlockSpec((64, 1), lambda i: (0, 0)),
            pl.BlockSpec((1, 1), lambda i: (0, 0)),
        ],
        out_specs=pl.BlockSpec((bm, 1), lambda i: (i, 0)),
        out_shape=jax.ShapeDtypeStruct((BATCH, 1), jnp.float32),
    )(cats4, nums, w1e, w1n, b1, w2, b2)


def kernel(cats, nums, emb_tables, W1, b1, W2, b2):
    tab_t = jnp.transpose(emb_tables, (0, 2, 1))
    tail = _tail(tab_t)
    ltab1, lcats = _reformat(tab_t, tail, cats.astype(jnp.int32))
    ltab = ltab1.reshape(N_FIELDS * VPAD, EMB_DIM)
    cats4 = _gather(lcats, ltab)
    w1e = W1[: N_FIELDS * EMB_DIM]
    w1n = W1[N_FIELDS * EMB_DIM :]
    return _mlp(cats4, nums, w1e, w1n, b1[None, :], W2, b2[None, :])


# speedup vs baseline: 1.0556x; 1.0556x over previous
import functools

import jax
import jax.numpy as jnp
from jax import lax
from jax.experimental import pallas as pl
from jax.experimental.pallas import tpu as pltpu
from jax.experimental.pallas import tpu_sc as plsc

N_FIELDS = 26
VOCAB = 100000
EMB_DIM = 16
NUM_DIM = 13
BATCH = 16384

VPAD = 100096
TAIL0 = 99968

NUM_WORKERS = 32
N_CHUNKS = 16
CHUNK = BATCH // N_CHUNKS
TASKS_PER_TILE = (N_FIELDS * N_CHUNKS) // NUM_WORKERS

BULK_CHUNKS = 97
TOT_BULK = N_FIELDS * BULK_CHUNKS
BULK_STEPS = (TOT_BULK + NUM_WORKERS - 1) // NUM_WORKERS


def _tail_body(tab_ref, out_ref):
    x = tab_ref[0]
    eye = jnp.eye(EMB_DIM, dtype=jnp.float32)
    y = jax.lax.dot_general(x, eye, (((0,), (0,)), ((), ())),
                            preferred_element_type=jnp.float32)
    y8 = y.reshape(16, 8, EMB_DIM)
    out_ref[...] = jnp.concatenate([y8[:, h, :] for h in range(8)], axis=1)


def _tail(tab_t):
    return pl.pallas_call(
        _tail_body,
        grid=(N_FIELDS,),
        in_specs=[pl.BlockSpec((1, EMB_DIM, 128),
                               lambda f: (f, 0, TAIL0 // 128))],
        out_specs=pl.BlockSpec((16, 128), lambda f: (f, 0)),
        out_shape=jax.ShapeDtypeStruct((N_FIELDS * 16, 128), jnp.float32),
    )(tab_t)


def _reformat_body(tab_hbm, tail_hbm, cats_hbm, ltab_hbm, lcats_hbm,
                   inb0, inb1, outb0, outb1, rowb, tailb,
                   sin0, sin1, sout0, sout1):
    wid = lax.axis_index("s") * 2 + lax.axis_index("c")
    lanes = lax.iota(jnp.int32, 16)

    def chunk_q(t):
        return jnp.minimum(t * NUM_WORKERS + wid, TOT_BULK - 1)

    def start_in(t, ib, sem):
        q = chunk_q(t)
        f = q // BULK_CHUNKS
        col0 = (q % BULK_CHUNKS) * 1024
        pltpu.async_copy(tab_hbm.at[f, :, pl.ds(col0, 1024)], ib, sem)

    def wait_in(sem):
        pltpu.make_async_copy(tab_hbm.at[0, :, pl.ds(0, 1024)], inb0,
                              sem).wait()

    def extract(ib, ob, width):
        @plsc.parallel_loop(0, width, unroll=16)
        def _(v):
            x = plsc.load_gather(ib, [lanes, jnp.full((16,), v, jnp.int32)])
            ob[pl.ds(v * EMB_DIM, EMB_DIM)] = x

    def start_out(t, ob, sem):
        q = chunk_q(t)
        f = q // BULK_CHUNKS
        col0 = (q % BULK_CHUNKS) * 1024
        dst = (f * VPAD + col0) * EMB_DIM
        pltpu.async_copy(ob, ltab_hbm.at[pl.ds(dst, 1024 * EMB_DIM)], sem)

    def wait_out(sem):
        pltpu.make_async_copy(outb0, ltab_hbm.at[pl.ds(0, 1024 * EMB_DIM)],
                              sem).wait()

    start_in(0, inb0, sin0)

    def phase(t, ib, ob, sin, sout, ib_next, sin_next):
        @pl.when(t + 1 < BULK_STEPS)
        def _():
            start_in(t + 1, ib_next, sin_next)

        wait_in(sin)

        @pl.when(t >= 2)
        def _():
            wait_out(sout)

        extract(ib, ob, 1024)
        start_out(t, ob, sout)

    def body(t, _):
        @pl.when(t % 2 == 0)
        def _():
            phase(t, inb0, outb0, sin0, sout0, inb1, sin1)

        @pl.when(t % 2 == 1)
        def _():
            phase(t, inb1, outb1, sin1, sout1, inb0, sin0)

        return 0

    lax.fori_loop(0, BULK_STEPS, body, 0)
    wait_out(sout0)
    wait_out(sout1)

    @pl.when(wid < N_FIELDS)
    def _():
        f = wid
        w = TAIL0 - BULK_CHUNKS * 1024
        pltpu.async_copy(tab_hbm.at[f, :, pl.ds(BULK_CHUNKS * 1024, w)],
                         inb0.at[:, pl.ds(0, w)], sin0)
        pltpu.make_async_copy(tab_hbm.at[0, :, pl.ds(0, w)],
                              inb0.at[:, pl.ds(0, w)], sin0).wait()
        extract(inb0, outb0, w)
        pltpu.async_copy(outb0.at[pl.ds(0, w * EMB_DIM)],
                         ltab_hbm.at[pl.ds((f * VPAD + BULK_CHUNKS * 1024)
                                           * EMB_DIM, w * EMB_DIM)], sout0)
        pltpu.sync_copy(tail_hbm.at[pl.ds(f * 16, 16)], tailb)

        def trow(v, _):
            x = plsc.load_gather(
                tailb, [jnp.full((16,), v // 8, jnp.int32),
                        (v % 8) * EMB_DIM + lanes])
            outb1[pl.ds(v * EMB_DIM, EMB_DIM)] = x
            return 0

        lax.fori_loop(0, 128, trow, 0, unroll=8)
        pltpu.async_copy(outb1.at[pl.ds(0, 128 * EMB_DIM)],
                         ltab_hbm.at[pl.ds((f * VPAD + TAIL0) * EMB_DIM,
                                           128 * EMB_DIM)], sout1)
        pltpu.sync_copy(cats_hbm.at[f], rowb)

        def add_off(k, _):
            rowb[pl.ds(k * 16, 16)] = rowb[pl.ds(k * 16, 16)] + f * VPAD
            return 0

        lax.fori_loop(0, BATCH // 16, add_off, 0, unroll=8)
        pltpu.sync_copy(rowb, lcats_hbm.at[pl.ds(f * BATCH, BATCH)])
        pltpu.make_async_copy(outb0.at[pl.ds(0, w * EMB_DIM)],
                              ltab_hbm.at[pl.ds(0, w * EMB_DIM)],
                              sout0).wait()
        pltpu.make_async_copy(outb1.at[pl.ds(0, 128 * EMB_DIM)],
                              ltab_hbm.at[pl.ds(0, 128 * EMB_DIM)],
                              sout1).wait()


_reformat = functools.partial(
    pl.kernel,
    out_type=(
        jax.ShapeDtypeStruct((N_FIELDS * VPAD * EMB_DIM,), jnp.float32),
        jax.ShapeDtypeStruct((N_FIELDS * BATCH,), jnp.int32),
    ),
    mesh=plsc.VectorSubcoreMesh(core_axis_name="c", subcore_axis_name="s"),
    scratch_types=[
        pltpu.VMEM((EMB_DIM, 1024), jnp.float32),
        pltpu.VMEM((EMB_DIM, 1024), jnp.float32),
        pltpu.VMEM((1024 * EMB_DIM,), jnp.float32),
        pltpu.VMEM((1024 * EMB_DIM,), jnp.float32),
        pltpu.VMEM((BATCH,), jnp.int32),
        pltpu.VMEM((16, 128), jnp.float32),
        pltpu.SemaphoreType.DMA,
        pltpu.SemaphoreType.DMA,
        pltpu.SemaphoreType.DMA,
        pltpu.SemaphoreType.DMA,
    ],
    compiler_params=pltpu.CompilerParams(needs_layout_passes=False),
)(_reformat_body)


def _gather_body(lcats_hbm, ltab_hbm, out_hbm, idx_v, rows_v, sem):
    wid = lax.axis_index("s") * 2 + lax.axis_index("c")

    def body(t, carry):
        task = wid * TASKS_PER_TILE + t
        f = task // N_CHUNKS
        c = task % N_CHUNKS
        base = c * CHUNK
        pltpu.sync_copy(lcats_hbm.at[pl.ds(f * BATCH + base, CHUNK)], idx_v)
        pltpu.async_copy(ltab_hbm.at[idx_v], rows_v, sem).wait()
        pltpu.sync_copy(rows_v,
                        out_hbm.at[f // 8, pl.ds(base, CHUNK),
                                   pl.ds((f % 8) * EMB_DIM, EMB_DIM)])
        return carry

    lax.fori_loop(0, TASKS_PER_TILE, body, 0)


_gather = functools.partial(
    pl.kernel,
    out_type=jax.ShapeDtypeStruct((4, BATCH, 128), jnp.float32),
    mesh=plsc.VectorSubcoreMesh(core_axis_name="c", subcore_axis_name="s"),
    scratch_types=[
        pltpu.VMEM((CHUNK,), jnp.int32),
        pltpu.VMEM((CHUNK, EMB_DIM), jnp.float32),
        pltpu.SemaphoreType.DMA,
    ],
    compiler_params=pltpu.CompilerParams(use_tc_tiling_on_sc=False),
)(_gather_body)


def _mlp_body(cat_ref, nums_ref, w1e_ref, w1n_ref, b1_ref, w2_ref, b2_ref,
              out_ref):
    x4 = cat_ref[...]
    w1e = w1e_ref[...]
    acc = nums_ref[...] @ w1n_ref[...] + b1_ref[...]
    for g in range(3):
        acc = acc + x4[g] @ w1e[g * 128:(g + 1) * 128]
    acc = acc + x4[3][:, :32] @ w1e[384:416]
    h = jnp.maximum(acc, 0.0)
    out_ref[...] = h @ w2_ref[...] + b2_ref[...]


def _mlp(cats4, nums, w1e, w1n, b1, w2, b2):
    bm = 2048
    grid = BATCH // bm
    return pl.pallas_call(
        _mlp_body,
        grid=(grid,),
        in_specs=[
            pl.BlockSpec((4, bm, 128), lambda i: (0, i, 0)),
            pl.BlockSpec((bm, NUM_DIM), lambda i: (i, 0)),
            pl.BlockSpec((N_FIELDS * EMB_DIM, 64), lambda i: (0, 0)),
            pl.BlockSpec((NUM_DIM, 64), lambda i: (0, 0)),
            pl.BlockSpec((1, 64), lambda i: (0, 0)),
            pl.BlockSpec((64, 1), lambda i: (0, 0)),
            pl.BlockSpec((1, 1), lambda i: (0, 0)),
        ],
        out_specs=pl.BlockSpec((bm, 1), lambda i: (i, 0)),
        out_shape=jax.ShapeDtypeStruct((BATCH, 1), jnp.float32),
    )(cats4, nums, w1e, w1n, b1, w2, b2)


def kernel(cats, nums, emb_tables, W1, b1, W2, b2):
    tab_t = jnp.transpose(emb_tables, (0, 2, 1))
    tail = _tail(tab_t)
    ltab1, lcats = _reformat(tab_t, tail, cats.astype(jnp.int32))
    ltab = ltab1.reshape(N_FIELDS * VPAD, EMB_DIM)
    cats4 = _gather(lcats, ltab)
    w1e = W1[: N_FIELDS * EMB_DIM]
    w1n = W1[N_FIELDS * EMB_DIM :]
    return _mlp(cats4, nums, w1e, w1n, b1[None, :], W2, b2[None, :])


# speedup vs baseline: 1.1283x; 1.0688x over previous
import functools

import jax
import jax.numpy as jnp
from jax import lax
from jax.experimental import pallas as pl
from jax.experimental.pallas import tpu as pltpu
from jax.experimental.pallas import tpu_sc as plsc

N_FIELDS = 26
VOCAB = 100000
EMB_DIM = 16
NUM_DIM = 13
BATCH = 16384

VPAD = 100096
TAIL0 = 99968

NUM_WORKERS = 32
N_CHUNKS = 16
CHUNK = BATCH // N_CHUNKS
TASKS_PER_TILE = (N_FIELDS * N_CHUNKS) // NUM_WORKERS

BULK_CHUNKS = 97
TOT_BULK = N_FIELDS * BULK_CHUNKS
BULK_STEPS = (TOT_BULK + NUM_WORKERS - 1) // NUM_WORKERS


def _tail_body(tab_ref, out_ref):
    x = tab_ref[0]
    eye = jnp.eye(EMB_DIM, dtype=jnp.float32)
    y = jax.lax.dot_general(x, eye, (((0,), (0,)), ((), ())),
                            preferred_element_type=jnp.float32)
    y8 = y.reshape(16, 8, EMB_DIM)
    out_ref[...] = jnp.concatenate([y8[:, h, :] for h in range(8)], axis=1)


def _tail(tab_t):
    return pl.pallas_call(
        _tail_body,
        grid=(N_FIELDS,),
        in_specs=[pl.BlockSpec((1, EMB_DIM, 128),
                               lambda f: (f, 0, TAIL0 // 128))],
        out_specs=pl.BlockSpec((16, 128), lambda f: (f, 0)),
        out_shape=jax.ShapeDtypeStruct((N_FIELDS * 16, 128), jnp.float32),
    )(tab_t)


def _reformat_body(tab_hbm, tail_hbm, cats_hbm, ltab_hbm, lcats_hbm,
                   inb0, inb1, outb0, outb1, rowb, tailb,
                   sin0, sin1, sout0, sout1):
    wid = lax.axis_index("s") * 2 + lax.axis_index("c")
    lanes = lax.iota(jnp.int32, 16)

    def chunk_q(t):
        return jnp.minimum(t * NUM_WORKERS + wid, TOT_BULK - 1)

    def start_in(t, ib, sem):
        q = chunk_q(t)
        f = q // BULK_CHUNKS
        col0 = (q % BULK_CHUNKS) * 1024
        pltpu.async_copy(tab_hbm.at[f, :, pl.ds(col0, 1024)], ib, sem)

    def wait_in(sem):
        pltpu.make_async_copy(tab_hbm.at[0, :, pl.ds(0, 1024)], inb0,
                              sem).wait()

    def extract(ib, ob, width):
        @plsc.parallel_loop(0, width, unroll=16,
                            carry=jnp.zeros((16,), jnp.int32))
        def _(v, col):
            x = plsc.load_gather(ib, [lanes, col])
            ob[pl.ds(v * EMB_DIM, EMB_DIM)] = x
            return col + 1

    def start_out(t, ob, sem):
        q = chunk_q(t)
        f = q // BULK_CHUNKS
        col0 = (q % BULK_CHUNKS) * 1024
        dst = (f * VPAD + col0) * EMB_DIM
        pltpu.async_copy(ob, ltab_hbm.at[pl.ds(dst, 1024 * EMB_DIM)], sem)

    def wait_out(sem):
        pltpu.make_async_copy(outb0, ltab_hbm.at[pl.ds(0, 1024 * EMB_DIM)],
                              sem).wait()

    start_in(0, inb0, sin0)

    def phase(t, ib, ob, sin, sout, ib_next, sin_next):
        @pl.when(t + 1 < BULK_STEPS)
        def _():
            start_in(t + 1, ib_next, sin_next)

        wait_in(sin)

        @pl.when(t >= 2)
        def _():
            wait_out(sout)

        extract(ib, ob, 1024)
        start_out(t, ob, sout)

    def body(t, _):
        @pl.when(t % 2 == 0)
        def _():
            phase(t, inb0, outb0, sin0, sout0, inb1, sin1)

        @pl.when(t % 2 == 1)
        def _():
            phase(t, inb1, outb1, sin1, sout1, inb0, sin0)

        return 0

    lax.fori_loop(0, BULK_STEPS, body, 0)
    wait_out(sout0)
    wait_out(sout1)

    @pl.when(wid < N_FIELDS)
    def _():
        f = wid
        w = TAIL0 - BULK_CHUNKS * 1024
        pltpu.async_copy(tab_hbm.at[f, :, pl.ds(BULK_CHUNKS * 1024, w)],
                         inb0.at[:, pl.ds(0, w)], sin0)
        pltpu.make_async_copy(tab_hbm.at[0, :, pl.ds(0, w)],
                              inb0.at[:, pl.ds(0, w)], sin0).wait()
        extract(inb0, outb0, w)
        pltpu.async_copy(outb0.at[pl.ds(0, w * EMB_DIM)],
                         ltab_hbm.at[pl.ds((f * VPAD + BULK_CHUNKS * 1024)
                                           * EMB_DIM, w * EMB_DIM)], sout0)
        pltpu.sync_copy(tail_hbm.at[pl.ds(f * 16, 16)], tailb)

        def trow(v, _):
            x = plsc.load_gather(
                tailb, [jnp.full((16,), v // 8, jnp.int32),
                        (v % 8) * EMB_DIM + lanes])
            outb1[pl.ds(v * EMB_DIM, EMB_DIM)] = x
            return 0

        lax.fori_loop(0, 128, trow, 0, unroll=8)
        pltpu.async_copy(outb1.at[pl.ds(0, 128 * EMB_DIM)],
                         ltab_hbm.at[pl.ds((f * VPAD + TAIL0) * EMB_DIM,
                                           128 * EMB_DIM)], sout1)
        pltpu.sync_copy(cats_hbm.at[f], rowb)

        def add_off(k, _):
            rowb[pl.ds(k * 16, 16)] = rowb[pl.ds(k * 16, 16)] + f * VPAD
            return 0

        lax.fori_loop(0, BATCH // 16, add_off, 0, unroll=8)
        pltpu.sync_copy(rowb, lcats_hbm.at[pl.ds(f * BATCH, BATCH)])
        pltpu.make_async_copy(outb0.at[pl.ds(0, w * EMB_DIM)],
                              ltab_hbm.at[pl.ds(0, w * EMB_DIM)],
                              sout0).wait()
        pltpu.make_async_copy(outb1.at[pl.ds(0, 128 * EMB_DIM)],
                              ltab_hbm.at[pl.ds(0, 128 * EMB_DIM)],
                              sout1).wait()


_reformat = functools.partial(
    pl.kernel,
    out_type=(
        jax.ShapeDtypeStruct((N_FIELDS * VPAD * EMB_DIM,), jnp.float32),
        jax.ShapeDtypeStruct((N_FIELDS * BATCH,), jnp.int32),
    ),
    mesh=plsc.VectorSubcoreMesh(core_axis_name="c", subcore_axis_name="s"),
    scratch_types=[
        pltpu.VMEM((EMB_DIM, 1024), jnp.float32),
        pltpu.VMEM((EMB_DIM, 1024), jnp.float32),
        pltpu.VMEM((1024 * EMB_DIM,), jnp.float32),
        pltpu.VMEM((1024 * EMB_DIM,), jnp.float32),
        pltpu.VMEM((BATCH,), jnp.int32),
        pltpu.VMEM((16, 128), jnp.float32),
        pltpu.SemaphoreType.DMA,
        pltpu.SemaphoreType.DMA,
        pltpu.SemaphoreType.DMA,
        pltpu.SemaphoreType.DMA,
    ],
    compiler_params=pltpu.CompilerParams(needs_layout_passes=False),
)(_reformat_body)


def _gather_body(lcats_hbm, ltab_hbm, out_hbm, idx_v, rows_v, sem):
    wid = lax.axis_index("s") * 2 + lax.axis_index("c")

    def body(t, carry):
        task = wid * TASKS_PER_TILE + t
        f = task // N_CHUNKS
        c = task % N_CHUNKS
        base = c * CHUNK
        pltpu.sync_copy(lcats_hbm.at[pl.ds(f * BATCH + base, CHUNK)], idx_v)
        pltpu.async_copy(ltab_hbm.at[idx_v], rows_v, sem).wait()
        pltpu.sync_copy(rows_v,
                        out_hbm.at[f // 8, pl.ds(base, CHUNK),
                                   pl.ds((f % 8) * EMB_DIM, EMB_DIM)])
        return carry

    lax.fori_loop(0, TASKS_PER_TILE, body, 0)


_gather = functools.partial(
    pl.kernel,
    out_type=jax.ShapeDtypeStruct((4, BATCH, 128), jnp.float32),
    mesh=plsc.VectorSubcoreMesh(core_axis_name="c", subcore_axis_name="s"),
    scratch_types=[
        pltpu.VMEM((CHUNK,), jnp.int32),
        pltpu.VMEM((CHUNK, EMB_DIM), jnp.float32),
        pltpu.SemaphoreType.DMA,
    ],
    compiler_params=pltpu.CompilerParams(use_tc_tiling_on_sc=False),
)(_gather_body)


def _mlp_body(cat_ref, nums_ref, w1e_ref, w1n_ref, b1_ref, w2_ref, b2_ref,
              out_ref):
    x4 = cat_ref[...]
    w1e = w1e_ref[...]
    acc = nums_ref[...] @ w1n_ref[...] + b1_ref[...]
    for g in range(3):
        acc = acc + x4[g] @ w1e[g * 128:(g + 1) * 128]
    acc = acc + x4[3][:, :32] @ w1e[384:416]
    h = jnp.maximum(acc, 0.0)
    out_ref[...] = h @ w2_ref[...] + b2_ref[...]


def _mlp(cats4, nums, w1e, w1n, b1, w2, b2):
    bm = 2048
    grid = BATCH // bm
    return pl.pallas_call(
        _mlp_body,
        grid=(grid,),
        in_specs=[
            pl.BlockSpec((4, bm, 128), lambda i: (0, i, 0)),
            pl.BlockSpec((bm, NUM_DIM), lambda i: (i, 0)),
            pl.BlockSpec((N_FIELDS * EMB_DIM, 64), lambda i: (0, 0)),
            pl.BlockSpec((NUM_DIM, 64), lambda i: (0, 0)),
            pl.BlockSpec((1, 64), lambda i: (0, 0)),
            pl.BlockSpec((64, 1), lambda i: (0, 0)),
            pl.BlockSpec((1, 1), lambda i: (0, 0)),
        ],
        out_specs=pl.BlockSpec((bm, 1), lambda i: (i, 0)),
        out_shape=jax.ShapeDtypeStruct((BATCH, 1), jnp.float32),
    )(cats4, nums, w1e, w1n, b1, w2, b2)


def kernel(cats, nums, emb_tables, W1, b1, W2, b2):
    tab_t = jnp.transpose(emb_tables, (0, 2, 1))
    tail = _tail(tab_t)
    ltab1, lcats = _reformat(tab_t, tail, cats.astype(jnp.int32))
    ltab = ltab1.reshape(N_FIELDS * VPAD, EMB_DIM)
    cats4 = _gather(lcats, ltab)
    w1e = W1[: N_FIELDS * EMB_DIM]
    w1n = W1[N_FIELDS * EMB_DIM :]
    return _mlp(cats4, nums, w1e, w1n, b1[None, :], W2, b2[None, :])


# speedup vs baseline: 1.1283x; 1.0000x over previous
import functools

import jax
import jax.numpy as jnp
from jax import lax
from jax.experimental import pallas as pl
from jax.experimental.pallas import tpu as pltpu
from jax.experimental.pallas import tpu_sc as plsc

N_FIELDS = 26
VOCAB = 100000
EMB_DIM = 16
NUM_DIM = 13
BATCH = 16384

VPAD = 100096
TAIL0 = 99968

NUM_WORKERS = 32
N_CHUNKS = 16
CHUNK = BATCH // N_CHUNKS
TASKS_PER_TILE = (N_FIELDS * N_CHUNKS) // NUM_WORKERS

BULK_CHUNKS = 97
TOT_BULK = N_FIELDS * BULK_CHUNKS
BULK_STEPS = (TOT_BULK + NUM_WORKERS - 1) // NUM_WORKERS


def _tail_body(tab_ref, out_ref):
    x = tab_ref[0]
    eye = jnp.eye(EMB_DIM, dtype=jnp.float32)
    y = jax.lax.dot_general(x, eye, (((0,), (0,)), ((), ())),
                            preferred_element_type=jnp.float32)
    y8 = y.reshape(16, 8, EMB_DIM)
    out_ref[...] = jnp.concatenate([y8[:, h, :] for h in range(8)], axis=1)


def _tail(tab_t):
    return pl.pallas_call(
        _tail_body,
        grid=(N_FIELDS,),
        in_specs=[pl.BlockSpec((1, EMB_DIM, 128),
                               lambda f: (f, 0, TAIL0 // 128))],
        out_specs=pl.BlockSpec((16, 128), lambda f: (f, 0)),
        out_shape=jax.ShapeDtypeStruct((N_FIELDS * 16, 128), jnp.float32),
    )(tab_t)


def _reformat_body(tab_hbm, tail_hbm, cats_hbm, ltab_hbm, lcats_hbm,
                   inb0, inb1, outb0, outb1, rowb, tailb,
                   sin0, sin1, sout0, sout1):
    wid = lax.axis_index("s") * 2 + lax.axis_index("c")
    lanes = lax.iota(jnp.int32, 16)

    def chunk_q(t):
        return jnp.minimum(t * NUM_WORKERS + wid, TOT_BULK - 1)

    def start_in(t, ib, sem):
        q = chunk_q(t)
        f = q // BULK_CHUNKS
        col0 = (q % BULK_CHUNKS) * 1024
        pltpu.async_copy(tab_hbm.at[f, :, pl.ds(col0, 1024)], ib, sem)

    def wait_in(sem):
        pltpu.make_async_copy(tab_hbm.at[0, :, pl.ds(0, 1024)], inb0,
                              sem).wait()

    def extract(ib, ob, width):
        @plsc.parallel_loop(0, width, unroll=8,
                            carry=jnp.zeros((16,), jnp.int32))
        def _(v, col):
            x = plsc.load_gather(ib, [lanes, col])
            ob[pl.ds(v * EMB_DIM, EMB_DIM)] = x
            return col + 1

    def start_out(t, ob, sem):
        q = chunk_q(t)
        f = q // BULK_CHUNKS
        col0 = (q % BULK_CHUNKS) * 1024
        dst = (f * VPAD + col0) * EMB_DIM
        pltpu.async_copy(ob, ltab_hbm.at[pl.ds(dst, 1024 * EMB_DIM)], sem)

    def wait_out(sem):
        pltpu.make_async_copy(outb0, ltab_hbm.at[pl.ds(0, 1024 * EMB_DIM)],
                              sem).wait()

    start_in(0, inb0, sin0)

    def phase(t, ib, ob, sin, sout, ib_next, sin_next):
        @pl.when(t + 1 < BULK_STEPS)
        def _():
            start_in(t + 1, ib_next, sin_next)

        wait_in(sin)

        @pl.when(t >= 2)
        def _():
            wait_out(sout)

        extract(ib, ob, 1024)
        start_out(t, ob, sout)

    def body(t, _):
        @pl.when(t % 2 == 0)
        def _():
            phase(t, inb0, outb0, sin0, sout0, inb1, sin1)

        @pl.when(t % 2 == 1)
        def _():
            phase(t, inb1, outb1, sin1, sout1, inb0, sin0)

        return 0

    lax.fori_loop(0, BULK_STEPS, body, 0)
    wait_out(sout0)
    wait_out(sout1)

    @pl.when(wid < N_FIELDS)
    def _():
        f = wid
        w = TAIL0 - BULK_CHUNKS * 1024
        pltpu.async_copy(tab_hbm.at[f, :, pl.ds(BULK_CHUNKS * 1024, w)],
                         inb0.at[:, pl.ds(0, w)], sin0)
        pltpu.make_async_copy(tab_hbm.at[0, :, pl.ds(0, w)],
                              inb0.at[:, pl.ds(0, w)], sin0).wait()
        extract(inb0, outb0, w)
        pltpu.async_copy(outb0.at[pl.ds(0, w * EMB_DIM)],
                         ltab_hbm.at[pl.ds((f * VPAD + BULK_CHUNKS * 1024)
                                           * EMB_DIM, w * EMB_DIM)], sout0)
        pltpu.sync_copy(tail_hbm.at[pl.ds(f * 16, 16)], tailb)

        def trow(v, _):
            x = plsc.load_gather(
                tailb, [jnp.full((16,), v // 8, jnp.int32),
                        (v % 8) * EMB_DIM + lanes])
            outb1[pl.ds(v * EMB_DIM, EMB_DIM)] = x
            return 0

        lax.fori_loop(0, 128, trow, 0, unroll=8)
        pltpu.async_copy(outb1.at[pl.ds(0, 128 * EMB_DIM)],
                         ltab_hbm.at[pl.ds((f * VPAD + TAIL0) * EMB_DIM,
                                           128 * EMB_DIM)], sout1)
        pltpu.sync_copy(cats_hbm.at[f], rowb)

        def add_off(k, _):
            rowb[pl.ds(k * 16, 16)] = rowb[pl.ds(k * 16, 16)] + f * VPAD
            return 0

        lax.fori_loop(0, BATCH // 16, add_off, 0, unroll=8)
        pltpu.sync_copy(rowb, lcats_hbm.at[pl.ds(f * BATCH, BATCH)])
        pltpu.make_async_copy(outb0.at[pl.ds(0, w * EMB_DIM)],
                              ltab_hbm.at[pl.ds(0, w * EMB_DIM)],
                              sout0).wait()
        pltpu.make_async_copy(outb1.at[pl.ds(0, 128 * EMB_DIM)],
                              ltab_hbm.at[pl.ds(0, 128 * EMB_DIM)],
                              sout1).wait()


_reformat = functools.partial(
    pl.kernel,
    out_type=(
        jax.ShapeDtypeStruct((N_FIELDS * VPAD * EMB_DIM,), jnp.float32),
        jax.ShapeDtypeStruct((N_FIELDS * BATCH,), jnp.int32),
    ),
    mesh=plsc.VectorSubcoreMesh(core_axis_name="c", subcore_axis_name="s"),
    scratch_types=[
        pltpu.VMEM((EMB_DIM, 1024), jnp.float32),
        pltpu.VMEM((EMB_DIM, 1024), jnp.float32),
        pltpu.VMEM((1024 * EMB_DIM,), jnp.float32),
        pltpu.VMEM((1024 * EMB_DIM,), jnp.float32),
        pltpu.VMEM((BATCH,), jnp.int32),
        pltpu.VMEM((16, 128), jnp.float32),
        pltpu.SemaphoreType.DMA,
        pltpu.SemaphoreType.DMA,
        pltpu.SemaphoreType.DMA,
        pltpu.SemaphoreType.DMA,
    ],
    compiler_params=pltpu.CompilerParams(needs_layout_passes=False),
)(_reformat_body)


def _gather_body(lcats_hbm, ltab_hbm, out_hbm, idx_v, rows_v, sem):
    wid = lax.axis_index("s") * 2 + lax.axis_index("c")

    def body(t, carry):
        task = wid * TASKS_PER_TILE + t
        f = task // N_CHUNKS
        c = task % N_CHUNKS
        base = c * CHUNK
        pltpu.sync_copy(lcats_hbm.at[pl.ds(f * BATCH + base, CHUNK)], idx_v)
        pltpu.async_copy(ltab_hbm.at[idx_v], rows_v, sem).wait()
        pltpu.sync_copy(rows_v,
                        out_hbm.at[f // 8, pl.ds(base, CHUNK),
                                   pl.ds((f % 8) * EMB_DIM, EMB_DIM)])
        return carry

    lax.fori_loop(0, TASKS_PER_TILE, body, 0)


_gather = functools.partial(
    pl.kernel,
    out_type=jax.ShapeDtypeStruct((4, BATCH, 128), jnp.float32),
    mesh=plsc.VectorSubcoreMesh(core_axis_name="c", subcore_axis_name="s"),
    scratch_types=[
        pltpu.VMEM((CHUNK,), jnp.int32),
        pltpu.VMEM((CHUNK, EMB_DIM), jnp.float32),
        pltpu.SemaphoreType.DMA,
    ],
    compiler_params=pltpu.CompilerParams(use_tc_tiling_on_sc=False),
)(_gather_body)


def _mlp_body(cat_ref, nums_ref, w1e_ref, w1n_ref, b1_ref, w2_ref, b2_ref,
              out_ref):
    x4 = cat_ref[...]
    w1e = w1e_ref[...]
    acc = nums_ref[...] @ w1n_ref[...] + b1_ref[...]
    for g in range(3):
        acc = acc + x4[g] @ w1e[g * 128:(g + 1) * 128]
    acc = acc + x4[3][:, :32] @ w1e[384:416]
    h = jnp.maximum(acc, 0.0)
    out_ref[...] = h @ w2_ref[...] + b2_ref[...]


def _mlp(cats4, nums, w1e, w1n, b1, w2, b2):
    bm = 2048
    grid = BATCH // bm
    return pl.pallas_call(
        _mlp_body,
        grid=(grid,),
        in_specs=[
            pl.BlockSpec((4, bm, 128), lambda i: (0, i, 0)),
            pl.BlockSpec((bm, NUM_DIM), lambda i: (i, 0)),
            pl.BlockSpec((N_FIELDS * EMB_DIM, 64), lambda i: (0, 0)),
            pl.BlockSpec((NUM_DIM, 64), lambda i: (0, 0)),
            pl.BlockSpec((1, 64), lambda i: (0, 0)),
            pl.BlockSpec((64, 1), lambda i: (0, 0)),
            pl.BlockSpec((1, 1), lambda i: (0, 0)),
        ],
        out_specs=pl.BlockSpec((bm, 1), lambda i: (i, 0)),
        out_shape=jax.ShapeDtypeStruct((BATCH, 1), jnp.float32),
    )(cats4, nums, w1e, w1n, b1, w2, b2)


def kernel(cats, nums, emb_tables, W1, b1, W2, b2):
    tab_t = jnp.transpose(emb_tables, (0, 2, 1))
    tail = _tail(tab_t)
    ltab1, lcats = _reformat(tab_t, tail, cats.astype(jnp.int32))
    ltab = ltab1.reshape(N_FIELDS * VPAD, EMB_DIM)
    cats4 = _gather(lcats, ltab)
    w1e = W1[: N_FIELDS * EMB_DIM]
    w1n = W1[N_FIELDS * EMB_DIM :]
    return _mlp(cats4, nums, w1e, w1n, b1[None, :], W2, b2[None, :])
